# Initial kernel scaffold; baseline (speedup 1.0000x reference)
import jax
import jax.numpy as jnp
from jax.experimental import pallas as pl


def kernel(t_in, y_in, alpha1, gamma1, alpha2, gamma2, k_smt, inds_r1, inds_p1, inds_r2, inds_p2, smt_reac, smt_prod, inds_surf, inds_mant):
    raise NotImplementedError("write your pallas kernel here")


# speedup vs baseline: 1.9109x; 1.9109x over previous
import jax
import jax.numpy as jnp
from jax import lax
from jax.experimental import pallas as pl
from jax.experimental.pallas import tpu as pltpu
from jax.experimental.pallas import tpu_sc as plsc

_B = 1024
_S = 512
_R1 = 4096
_R2 = 8192
_RS = 1024
_HALF = _RS // 2
_SURF_LO, _SURF_HI = 256, 384
_MANT_LO, _MANT_HI = 384, 512
_NSURF = _SURF_HI - _SURF_LO
_LANES = 16
_NC, _NS = 2, 16
_NW = _NC * _NS
_PASSES = _B // (_NW * _LANES)
_EPS = 1e-10


def _sc_body(yT, t_in, r1, p1, a1, g1, r2a, r2b, p2, a2, g2,
             sreac, sprod, ksmt, outT,
             yT_v, gain_v, loss_v, r1_v, p1_v, a1_v, g1_v,
             r2a_v, r2b_v, p2_v, a2_v, g2_v, sreac_v, sprod_v, ksmt_v, t_v):
    wid = lax.axis_index("s") * _NC + lax.axis_index("c")
    pltpu.sync_copy(r1, r1_v)
    pltpu.sync_copy(p1, p1_v)
    pltpu.sync_copy(a1, a1_v)
    pltpu.sync_copy(g1, g1_v)
    pltpu.sync_copy(r2a, r2a_v)
    pltpu.sync_copy(r2b, r2b_v)
    pltpu.sync_copy(p2, p2_v)
    pltpu.sync_copy(a2, a2_v)
    pltpu.sync_copy(g2, g2_v)
    pltpu.sync_copy(sreac, sreac_v)
    pltpu.sync_copy(sprod, sprod_v)
    pltpu.sync_copy(ksmt, ksmt_v)

    zero = jnp.zeros((_LANES,), jnp.float32)

    for pss in range(_PASSES):
        base = (pss * _NW + wid) * _LANES
        pltpu.sync_copy(yT.at[:, pl.ds(base, _LANES)], yT_v)
        pltpu.sync_copy(t_in.at[pl.ds(base, _LANES)], t_v)

        def zero_loop(s, _):
            gain_v[s] = zero
            loss_v[s] = zero
            return 0
        lax.fori_loop(0, _S, zero_loop, 0)

        ninvt = -1.0 / t_v[...]

        def r1_group(g, _):
            b16 = g * _LANES
            av = a1_v[pl.ds(b16, _LANES)]
            gv = g1_v[pl.ds(b16, _LANES)]
            riv = r1_v[pl.ds(b16, _LANES)]
            piv = p1_v[pl.ds(b16, _LANES)]
            for j in range(_LANES):
                rate = av[j] * jnp.exp(gv[j] * ninvt)
                ri = riv[j]
                term = rate * yT_v[ri]
                plsc.addupdate(gain_v.at[piv[j]], term)
                plsc.addupdate(loss_v.at[ri], term)
            return 0
        lax.fori_loop(0, _R1 // _LANES, r1_group, 0)

        def r2_group(g, _):
            b16 = g * _LANES
            av = a2_v[pl.ds(b16, _LANES)]
            gv = g2_v[pl.ds(b16, _LANES)]
            rav = r2a_v[pl.ds(b16, _LANES)]
            rbv = r2b_v[pl.ds(b16, _LANES)]
            pv = p2_v[pl.ds(b16, _LANES)]
            for j in range(_LANES):
                rate = av[j] * jnp.exp(gv[j] * ninvt)
                ra = rav[j]
                rb = rbv[j]
                term = rate * yT_v[ra] * yT_v[rb]
                plsc.addupdate(gain_v.at[pv[j]], term)
                plsc.addupdate(loss_v.at[ra], term)
                plsc.addupdate(loss_v.at[rb], term)
            return 0
        lax.fori_loop(0, _R2 // _LANES, r2_group, 0)

        def surf_loop(i, carry):
            sg, sl, ns, nm = carry
            sg = sg + gain_v[_SURF_LO + i]
            sl = sl + loss_v[_SURF_LO + i]
            ns = ns + yT_v[_SURF_LO + i]
            nm = nm + yT_v[_MANT_LO + i]
            return sg, sl, ns, nm
        sg, sl, ns, nm = lax.fori_loop(0, _NSURF, surf_loop,
                                       (zero, zero, zero, zero))

        sm_coef = sg / (ns + _EPS)
        ms_coef = sl / (nm + _EPS)

        def smt_sm_group(g, _):
            b16 = g * _LANES
            kv = ksmt_v[pl.ds(b16, _LANES)]
            rv = sreac_v[pl.ds(b16, _LANES)]
            pv = sprod_v[pl.ds(b16, _LANES)]
            for j in range(_LANES):
                rate = kv[j] * sm_coef
                rr = rv[j]
                term = rate * yT_v[rr]
                plsc.addupdate(gain_v.at[pv[j]], term)
                plsc.addupdate(loss_v.at[rr], term)
            return 0
        lax.fori_loop(0, _HALF // _LANES, smt_sm_group, 0)

        def smt_ms_group(g, _):
            b16 = _HALF + g * _LANES
            kv = ksmt_v[pl.ds(b16, _LANES)]
            rv = sreac_v[pl.ds(b16, _LANES)]
            pv = sprod_v[pl.ds(b16, _LANES)]
            for j in range(_LANES):
                rate = kv[j] * ms_coef
                rr = rv[j]
                term = rate * yT_v[rr]
                plsc.addupdate(gain_v.at[pv[j]], term)
                plsc.addupdate(loss_v.at[rr], term)
            return 0
        lax.fori_loop(0, _HALF // _LANES, smt_ms_group, 0)

        def fin_loop(s, _):
            gain_v[s] = gain_v[s] - loss_v[s]
            return 0
        lax.fori_loop(0, _S, fin_loop, 0)

        pltpu.sync_copy(gain_v, outT.at[:, pl.ds(base, _LANES)])


_sc_call = pl.kernel(
    _sc_body,
    out_type=jax.ShapeDtypeStruct((_S, _B), jnp.float32),
    mesh=plsc.VectorSubcoreMesh(core_axis_name="c", subcore_axis_name="s"),
    compiler_params=pltpu.CompilerParams(use_tc_tiling_on_sc=False),
    scratch_types=[
        pltpu.VMEM((_S, _LANES), jnp.float32),
        pltpu.VMEM((_S, _LANES), jnp.float32),
        pltpu.VMEM((_S, _LANES), jnp.float32),
        pltpu.VMEM((_R1,), jnp.int32),
        pltpu.VMEM((_R1,), jnp.int32),
        pltpu.VMEM((_R1,), jnp.float32),
        pltpu.VMEM((_R1,), jnp.float32),
        pltpu.VMEM((_R2,), jnp.int32),
        pltpu.VMEM((_R2,), jnp.int32),
        pltpu.VMEM((_R2,), jnp.int32),
        pltpu.VMEM((_R2,), jnp.float32),
        pltpu.VMEM((_R2,), jnp.float32),
        pltpu.VMEM((_RS,), jnp.int32),
        pltpu.VMEM((_RS,), jnp.int32),
        pltpu.VMEM((_RS,), jnp.float32),
        pltpu.VMEM((_LANES,), jnp.float32),
    ],
)


def kernel(t_in, y_in, alpha1, gamma1, alpha2, gamma2, k_smt,
           inds_r1, inds_p1, inds_r2, inds_p2, smt_reac, smt_prod,
           inds_surf, inds_mant):
    del inds_surf, inds_mant
    i32 = jnp.int32
    outT = _sc_call(
        y_in.T, t_in,
        inds_r1.astype(i32), inds_p1.astype(i32), alpha1, gamma1,
        inds_r2[:, 0].astype(i32), inds_r2[:, 1].astype(i32),
        inds_p2.astype(i32), alpha2, gamma2,
        smt_reac.astype(i32), smt_prod.astype(i32), k_smt,
    )
    return outT.T


# speedup vs baseline: 3.6270x; 1.8981x over previous
import jax
import jax.numpy as jnp
from jax import lax
from jax.experimental import pallas as pl
from jax.experimental.pallas import tpu as pltpu
from jax.experimental.pallas import tpu_sc as plsc

_B = 1024
_S = 512
_R1 = 4096
_R2 = 8192
_RS = 1024
_HALF = _RS // 2
_SURF_LO, _SURF_HI = 256, 384
_MANT_LO, _MANT_HI = 384, 512
_NSURF = _SURF_HI - _SURF_LO
_LANES = 16
_NC, _NS = 2, 16
_NW = _NC * _NS
_PASSES = _B // (_NW * _LANES)
_EPS = 1e-10


def _sc_body(yT, t_in, r1, p1, a1, g1, r2a, r2b, p2, a2, g2,
             sreac, sprod, ksmt, outT,
             yT_v, gain_v, loss_v, r1_v, p1_v, a1_v, g1_v,
             r2a_v, r2b_v, p2_v, a2_v, g2_v, sreac_v, sprod_v, ksmt_v, t_v):
    wid = lax.axis_index("s") * _NC + lax.axis_index("c")
    pltpu.sync_copy(r1, r1_v)
    pltpu.sync_copy(p1, p1_v)
    pltpu.sync_copy(a1, a1_v)
    pltpu.sync_copy(g1, g1_v)
    pltpu.sync_copy(r2a, r2a_v)
    pltpu.sync_copy(r2b, r2b_v)
    pltpu.sync_copy(p2, p2_v)
    pltpu.sync_copy(a2, a2_v)
    pltpu.sync_copy(g2, g2_v)
    pltpu.sync_copy(sreac, sreac_v)
    pltpu.sync_copy(sprod, sprod_v)
    pltpu.sync_copy(ksmt, ksmt_v)

    zero = jnp.zeros((_LANES,), jnp.float32)

    for pss in range(_PASSES):
        base = (pss * _NW + wid) * _LANES
        pltpu.sync_copy(yT.at[:, pl.ds(base, _LANES)], yT_v)
        pltpu.sync_copy(t_in.at[pl.ds(base, _LANES)], t_v)

        @plsc.parallel_loop(0, _S)
        def zero_loop(s):
            gain_v[s] = zero
            loss_v[s] = zero

        ninvt = -1.0 / t_v[...]

        @plsc.parallel_loop(0, _R1 // _LANES)
        def r1_group(g):
            b16 = g * _LANES
            av = a1_v[pl.ds(b16, _LANES)]
            gv = g1_v[pl.ds(b16, _LANES)]
            riv = r1_v[pl.ds(b16, _LANES)]
            piv = p1_v[pl.ds(b16, _LANES)]
            for j in range(_LANES):
                rate = av[j] * jnp.exp(gv[j] * ninvt)
                ri = riv[j]
                term = rate * yT_v[ri]
                plsc.addupdate(gain_v.at[piv[j]], term)
                plsc.addupdate(loss_v.at[ri], term)

        @plsc.parallel_loop(0, _R2 // _LANES)
        def r2_group(g):
            b16 = g * _LANES
            av = a2_v[pl.ds(b16, _LANES)]
            gv = g2_v[pl.ds(b16, _LANES)]
            rav = r2a_v[pl.ds(b16, _LANES)]
            rbv = r2b_v[pl.ds(b16, _LANES)]
            pv = p2_v[pl.ds(b16, _LANES)]
            for j in range(_LANES):
                rate = av[j] * jnp.exp(gv[j] * ninvt)
                ra = rav[j]
                rb = rbv[j]
                term = rate * yT_v[ra] * yT_v[rb]
                plsc.addupdate(gain_v.at[pv[j]], term)
                plsc.addupdate(loss_v.at[ra], term)
                plsc.addupdate(loss_v.at[rb], term)

        def surf_loop(i, carry):
            sg, sl, ns, nm = carry
            sg = sg + gain_v[_SURF_LO + i]
            sl = sl + loss_v[_SURF_LO + i]
            ns = ns + yT_v[_SURF_LO + i]
            nm = nm + yT_v[_MANT_LO + i]
            return sg, sl, ns, nm
        sg, sl, ns, nm = lax.fori_loop(0, _NSURF, surf_loop,
                                       (zero, zero, zero, zero))

        sm_coef = sg / (ns + _EPS)
        ms_coef = sl / (nm + _EPS)

        @plsc.parallel_loop(0, _HALF // _LANES)
        def smt_sm_group(g):
            b16 = g * _LANES
            kv = ksmt_v[pl.ds(b16, _LANES)]
            rv = sreac_v[pl.ds(b16, _LANES)]
            pv = sprod_v[pl.ds(b16, _LANES)]
            for j in range(_LANES):
                rate = kv[j] * sm_coef
                rr = rv[j]
                term = rate * yT_v[rr]
                plsc.addupdate(gain_v.at[pv[j]], term)
                plsc.addupdate(loss_v.at[rr], term)

        @plsc.parallel_loop(0, _HALF // _LANES)
        def smt_ms_group(g):
            b16 = _HALF + g * _LANES
            kv = ksmt_v[pl.ds(b16, _LANES)]
            rv = sreac_v[pl.ds(b16, _LANES)]
            pv = sprod_v[pl.ds(b16, _LANES)]
            for j in range(_LANES):
                rate = kv[j] * ms_coef
                rr = rv[j]
                term = rate * yT_v[rr]
                plsc.addupdate(gain_v.at[pv[j]], term)
                plsc.addupdate(loss_v.at[rr], term)

        @plsc.parallel_loop(0, _S)
        def fin_loop(s):
            gain_v[s] = gain_v[s] - loss_v[s]

        pltpu.sync_copy(gain_v, outT.at[:, pl.ds(base, _LANES)])


_sc_call = pl.kernel(
    _sc_body,
    out_type=jax.ShapeDtypeStruct((_S, _B), jnp.float32),
    mesh=plsc.VectorSubcoreMesh(core_axis_name="c", subcore_axis_name="s"),
    compiler_params=pltpu.CompilerParams(use_tc_tiling_on_sc=False),
    scratch_types=[
        pltpu.VMEM((_S, _LANES), jnp.float32),
        pltpu.VMEM((_S, _LANES), jnp.float32),
        pltpu.VMEM((_S, _LANES), jnp.float32),
        pltpu.VMEM((_R1,), jnp.int32),
        pltpu.VMEM((_R1,), jnp.int32),
        pltpu.VMEM((_R1,), jnp.float32),
        pltpu.VMEM((_R1,), jnp.float32),
        pltpu.VMEM((_R2,), jnp.int32),
        pltpu.VMEM((_R2,), jnp.int32),
        pltpu.VMEM((_R2,), jnp.int32),
        pltpu.VMEM((_R2,), jnp.float32),
        pltpu.VMEM((_R2,), jnp.float32),
        pltpu.VMEM((_RS,), jnp.int32),
        pltpu.VMEM((_RS,), jnp.int32),
        pltpu.VMEM((_RS,), jnp.float32),
        pltpu.VMEM((_LANES,), jnp.float32),
    ],
)


def kernel(t_in, y_in, alpha1, gamma1, alpha2, gamma2, k_smt,
           inds_r1, inds_p1, inds_r2, inds_p2, smt_reac, smt_prod,
           inds_surf, inds_mant):
    del inds_surf, inds_mant
    i32 = jnp.int32
    outT = _sc_call(
        y_in.T, t_in,
        inds_r1.astype(i32), inds_p1.astype(i32), alpha1, gamma1,
        inds_r2[:, 0].astype(i32), inds_r2[:, 1].astype(i32),
        inds_p2.astype(i32), alpha2, gamma2,
        smt_reac.astype(i32), smt_prod.astype(i32), k_smt,
    )
    return outT.T


# speedup vs baseline: 3.6704x; 1.0120x over previous
import jax
import jax.numpy as jnp
from jax import lax
from jax.experimental import pallas as pl
from jax.experimental.pallas import tpu as pltpu
from jax.experimental.pallas import tpu_sc as plsc

_B = 1024
_S = 512
_R1 = 4096
_R2 = 8192
_RS = 1024
_HALF = _RS // 2
_SURF_LO, _SURF_HI = 256, 384
_MANT_LO, _MANT_HI = 384, 512
_NSURF = _SURF_HI - _SURF_LO
_LANES = 16
_NC, _NS = 2, 16
_NW = _NC * _NS
_PASSES = _B // (_NW * _LANES)
_EPS = 1e-10


def _sc_body(yT, t_in, pk1, a1, g1, pk2, a2, g2, pks, ksmt, outT,
             yT_v, gain_v, loss_v, pk1_v, a1_v, g1_v,
             pk2_v, a2_v, g2_v, pks_v, ksmt_v, t_v):
    wid = lax.axis_index("s") * _NC + lax.axis_index("c")
    pltpu.sync_copy(pk1, pk1_v)
    pltpu.sync_copy(a1, a1_v)
    pltpu.sync_copy(g1, g1_v)
    pltpu.sync_copy(pk2, pk2_v)
    pltpu.sync_copy(a2, a2_v)
    pltpu.sync_copy(g2, g2_v)
    pltpu.sync_copy(pks, pks_v)
    pltpu.sync_copy(ksmt, ksmt_v)

    zero = jnp.zeros((_LANES,), jnp.float32)

    for pss in range(_PASSES):
        base = (pss * _NW + wid) * _LANES
        pltpu.sync_copy(yT.at[:, pl.ds(base, _LANES)], yT_v)
        pltpu.sync_copy(t_in.at[pl.ds(base, _LANES)], t_v)

        @plsc.parallel_loop(0, _S)
        def zero_loop(s):
            gain_v[s] = zero
            loss_v[s] = zero

        ninvt = -1.0 / t_v[...]

        @plsc.parallel_loop(0, _R1 // _LANES)
        def r1_group(g):
            b16 = g * _LANES
            av = a1_v[pl.ds(b16, _LANES)]
            gv = g1_v[pl.ds(b16, _LANES)]
            pkv = pk1_v[pl.ds(b16, _LANES)]
            for j in range(_LANES):
                rate = av[j] * jnp.exp(gv[j] * ninvt)
                pk = pkv[j]
                ri = pk & 0x3FF
                term = rate * yT_v[ri]
                plsc.addupdate(gain_v.at[pk >> 16], term)
                plsc.addupdate(loss_v.at[ri], term)

        @plsc.parallel_loop(0, _R2 // _LANES)
        def r2_group(g):
            b16 = g * _LANES
            av = a2_v[pl.ds(b16, _LANES)]
            gv = g2_v[pl.ds(b16, _LANES)]
            pkv = pk2_v[pl.ds(b16, _LANES)]
            for j in range(_LANES):
                rate = av[j] * jnp.exp(gv[j] * ninvt)
                pk = pkv[j]
                ra = pk & 0x3FF
                rb = (pk >> 10) & 0x3FF
                term = rate * yT_v[ra] * yT_v[rb]
                plsc.addupdate(gain_v.at[pk >> 20], term)
                plsc.addupdate(loss_v.at[ra], term)
                plsc.addupdate(loss_v.at[rb], term)

        def surf_loop(i, carry):
            sg, sl, ns, nm = carry
            sg = sg + gain_v[_SURF_LO + i]
            sl = sl + loss_v[_SURF_LO + i]
            ns = ns + yT_v[_SURF_LO + i]
            nm = nm + yT_v[_MANT_LO + i]
            return sg, sl, ns, nm
        sg, sl, ns, nm = lax.fori_loop(0, _NSURF, surf_loop,
                                       (zero, zero, zero, zero))

        sm_coef = sg / (ns + _EPS)
        ms_coef = sl / (nm + _EPS)

        @plsc.parallel_loop(0, _HALF // _LANES)
        def smt_sm_group(g):
            b16 = g * _LANES
            kv = ksmt_v[pl.ds(b16, _LANES)]
            pkv = pks_v[pl.ds(b16, _LANES)]
            for j in range(_LANES):
                rate = kv[j] * sm_coef
                pk = pkv[j]
                rr = pk & 0x3FF
                term = rate * yT_v[rr]
                plsc.addupdate(gain_v.at[pk >> 16], term)
                plsc.addupdate(loss_v.at[rr], term)

        @plsc.parallel_loop(0, _HALF // _LANES)
        def smt_ms_group(g):
            b16 = _HALF + g * _LANES
            kv = ksmt_v[pl.ds(b16, _LANES)]
            pkv = pks_v[pl.ds(b16, _LANES)]
            for j in range(_LANES):
                rate = kv[j] * ms_coef
                pk = pkv[j]
                rr = pk & 0x3FF
                term = rate * yT_v[rr]
                plsc.addupdate(gain_v.at[pk >> 16], term)
                plsc.addupdate(loss_v.at[rr], term)

        @plsc.parallel_loop(0, _S)
        def fin_loop(s):
            gain_v[s] = gain_v[s] - loss_v[s]

        pltpu.sync_copy(gain_v, outT.at[:, pl.ds(base, _LANES)])


_sc_call = pl.kernel(
    _sc_body,
    out_type=jax.ShapeDtypeStruct((_S, _B), jnp.float32),
    mesh=plsc.VectorSubcoreMesh(core_axis_name="c", subcore_axis_name="s"),
    compiler_params=pltpu.CompilerParams(use_tc_tiling_on_sc=False),
    scratch_types=[
        pltpu.VMEM((_S, _LANES), jnp.float32),
        pltpu.VMEM((_S, _LANES), jnp.float32),
        pltpu.VMEM((_S, _LANES), jnp.float32),
        pltpu.VMEM((_R1,), jnp.int32),
        pltpu.VMEM((_R1,), jnp.float32),
        pltpu.VMEM((_R1,), jnp.float32),
        pltpu.VMEM((_R2,), jnp.int32),
        pltpu.VMEM((_R2,), jnp.float32),
        pltpu.VMEM((_R2,), jnp.float32),
        pltpu.VMEM((_RS,), jnp.int32),
        pltpu.VMEM((_RS,), jnp.float32),
        pltpu.VMEM((_LANES,), jnp.float32),
    ],
)


def kernel(t_in, y_in, alpha1, gamma1, alpha2, gamma2, k_smt,
           inds_r1, inds_p1, inds_r2, inds_p2, smt_reac, smt_prod,
           inds_surf, inds_mant):
    del inds_surf, inds_mant
    i32 = jnp.int32
    r1 = inds_r1.astype(i32)
    p1 = inds_p1.astype(i32)
    r2a = inds_r2[:, 0].astype(i32)
    r2b = inds_r2[:, 1].astype(i32)
    p2 = inds_p2.astype(i32)
    sre = smt_reac.astype(i32)
    spr = smt_prod.astype(i32)
    pk1 = r1 | (p1 << 16)
    pk2 = r2a | (r2b << 10) | (p2 << 20)
    pks = sre | (spr << 16)
    outT = _sc_call(y_in.T, t_in, pk1, alpha1, gamma1,
                    pk2, alpha2, gamma2, pks, k_smt)
    return outT.T


# speedup vs baseline: 3.9659x; 1.0805x over previous
import jax
import jax.numpy as jnp
from jax import lax
from jax.experimental import pallas as pl
from jax.experimental.pallas import tpu as pltpu
from jax.experimental.pallas import tpu_sc as plsc

_B = 1024
_S = 512
_R1 = 4096
_R2 = 8192
_RS = 1024
_HALF = _RS // 2
_SURF_LO, _SURF_HI = 256, 384
_MANT_LO, _MANT_HI = 384, 512
_NSURF = _SURF_HI - _SURF_LO
_LANES = 16
_NC, _NS = 2, 16
_NW = _NC * _NS
_EPS = 1e-10


def _sc_body(yT, t_in, pk1, la1, g1, pk2, la2, g2, pks, ksmt, outT,
             yTa_v, yTb_v, gaina_v, gainb_v, lossa_v, lossb_v,
             pk1_v, la1_v, g1_v, pk2_v, la2_v, g2_v, pks_v, ksmt_v,
             ta_v, tb_v):
    wid = lax.axis_index("s") * _NC + lax.axis_index("c")
    base_a = wid * _LANES
    base_b = base_a + _NW * _LANES
    pltpu.sync_copy(pk1, pk1_v)
    pltpu.sync_copy(la1, la1_v)
    pltpu.sync_copy(g1, g1_v)
    pltpu.sync_copy(pk2, pk2_v)
    pltpu.sync_copy(la2, la2_v)
    pltpu.sync_copy(g2, g2_v)
    pltpu.sync_copy(pks, pks_v)
    pltpu.sync_copy(ksmt, ksmt_v)
    pltpu.sync_copy(yT.at[:, pl.ds(base_a, _LANES)], yTa_v)
    pltpu.sync_copy(yT.at[:, pl.ds(base_b, _LANES)], yTb_v)
    pltpu.sync_copy(t_in.at[pl.ds(base_a, _LANES)], ta_v)
    pltpu.sync_copy(t_in.at[pl.ds(base_b, _LANES)], tb_v)

    zero = jnp.zeros((_LANES,), jnp.float32)

    @plsc.parallel_loop(0, _S)
    def zero_loop(s):
        gaina_v[s] = zero
        gainb_v[s] = zero
        lossa_v[s] = zero
        lossb_v[s] = zero

    ninvt_a = -1.0 / ta_v[...]
    ninvt_b = -1.0 / tb_v[...]

    @plsc.parallel_loop(0, _R1 // _LANES)
    def r1_group(g):
        b16 = g * _LANES
        lav = la1_v[pl.ds(b16, _LANES)]
        gv = g1_v[pl.ds(b16, _LANES)]
        pkv = pk1_v[pl.ds(b16, _LANES)]
        for j in range(_LANES):
            gj = gv[j]
            lj = lav[j]
            ea = jnp.exp(gj * ninvt_a + lj)
            eb = jnp.exp(gj * ninvt_b + lj)
            pk = pkv[j]
            ri = pk & 0x3FF
            pp = pk >> 16
            ta = ea * yTa_v[ri]
            tb = eb * yTb_v[ri]
            plsc.addupdate(gaina_v.at[pp], ta)
            plsc.addupdate(gainb_v.at[pp], tb)
            plsc.addupdate(lossa_v.at[ri], ta)
            plsc.addupdate(lossb_v.at[ri], tb)

    @plsc.parallel_loop(0, _R2 // _LANES)
    def r2_group(g):
        b16 = g * _LANES
        lav = la2_v[pl.ds(b16, _LANES)]
        gv = g2_v[pl.ds(b16, _LANES)]
        pkv = pk2_v[pl.ds(b16, _LANES)]
        for j in range(_LANES):
            gj = gv[j]
            lj = lav[j]
            ea = jnp.exp(gj * ninvt_a + lj)
            eb = jnp.exp(gj * ninvt_b + lj)
            pk = pkv[j]
            ra = pk & 0x3FF
            rb = (pk >> 10) & 0x3FF
            pp = pk >> 20
            ta = ea * yTa_v[ra] * yTa_v[rb]
            tb = eb * yTb_v[ra] * yTb_v[rb]
            plsc.addupdate(gaina_v.at[pp], ta)
            plsc.addupdate(gainb_v.at[pp], tb)
            plsc.addupdate(lossa_v.at[ra], ta)
            plsc.addupdate(lossb_v.at[ra], tb)
            plsc.addupdate(lossa_v.at[rb], ta)
            plsc.addupdate(lossb_v.at[rb], tb)

    def surf_loop(i, carry):
        sga, sla, nsa, nma, sgb, slb, nsb, nmb = carry
        sga = sga + gaina_v[_SURF_LO + i]
        sla = sla + lossa_v[_SURF_LO + i]
        nsa = nsa + yTa_v[_SURF_LO + i]
        nma = nma + yTa_v[_MANT_LO + i]
        sgb = sgb + gainb_v[_SURF_LO + i]
        slb = slb + lossb_v[_SURF_LO + i]
        nsb = nsb + yTb_v[_SURF_LO + i]
        nmb = nmb + yTb_v[_MANT_LO + i]
        return sga, sla, nsa, nma, sgb, slb, nsb, nmb
    sga, sla, nsa, nma, sgb, slb, nsb, nmb = lax.fori_loop(
        0, _NSURF, surf_loop, (zero,) * 8)

    sm_a = sga / (nsa + _EPS)
    ms_a = sla / (nma + _EPS)
    sm_b = sgb / (nsb + _EPS)
    ms_b = slb / (nmb + _EPS)

    def _smt_loop(coef_a, coef_b, off):
        @plsc.parallel_loop(0, _HALF // _LANES)
        def smt_group(g):
            b16 = off + g * _LANES
            kv = ksmt_v[pl.ds(b16, _LANES)]
            pkv = pks_v[pl.ds(b16, _LANES)]
            for j in range(_LANES):
                kj = kv[j]
                pk = pkv[j]
                rr = pk & 0x3FF
                pp = pk >> 16
                ta = kj * coef_a * yTa_v[rr]
                tb = kj * coef_b * yTb_v[rr]
                plsc.addupdate(gaina_v.at[pp], ta)
                plsc.addupdate(gainb_v.at[pp], tb)
                plsc.addupdate(lossa_v.at[rr], ta)
                plsc.addupdate(lossb_v.at[rr], tb)

    _smt_loop(sm_a, sm_b, 0)
    _smt_loop(ms_a, ms_b, _HALF)

    @plsc.parallel_loop(0, _S)
    def fin_loop(s):
        gaina_v[s] = gaina_v[s] - lossa_v[s]
        gainb_v[s] = gainb_v[s] - lossb_v[s]

    pltpu.sync_copy(gaina_v, outT.at[:, pl.ds(base_a, _LANES)])
    pltpu.sync_copy(gainb_v, outT.at[:, pl.ds(base_b, _LANES)])


_sc_call = pl.kernel(
    _sc_body,
    out_type=jax.ShapeDtypeStruct((_S, _B), jnp.float32),
    mesh=plsc.VectorSubcoreMesh(core_axis_name="c", subcore_axis_name="s"),
    compiler_params=pltpu.CompilerParams(use_tc_tiling_on_sc=False),
    scratch_types=[
        pltpu.VMEM((_S, _LANES), jnp.float32),
        pltpu.VMEM((_S, _LANES), jnp.float32),
        pltpu.VMEM((_S, _LANES), jnp.float32),
        pltpu.VMEM((_S, _LANES), jnp.float32),
        pltpu.VMEM((_S, _LANES), jnp.float32),
        pltpu.VMEM((_S, _LANES), jnp.float32),
        pltpu.VMEM((_R1,), jnp.int32),
        pltpu.VMEM((_R1,), jnp.float32),
        pltpu.VMEM((_R1,), jnp.float32),
        pltpu.VMEM((_R2,), jnp.int32),
        pltpu.VMEM((_R2,), jnp.float32),
        pltpu.VMEM((_R2,), jnp.float32),
        pltpu.VMEM((_RS,), jnp.int32),
        pltpu.VMEM((_RS,), jnp.float32),
        pltpu.VMEM((_LANES,), jnp.float32),
        pltpu.VMEM((_LANES,), jnp.float32),
    ],
)


def kernel(t_in, y_in, alpha1, gamma1, alpha2, gamma2, k_smt,
           inds_r1, inds_p1, inds_r2, inds_p2, smt_reac, smt_prod,
           inds_surf, inds_mant):
    del inds_surf, inds_mant
    i32 = jnp.int32
    r1 = inds_r1.astype(i32)
    p1 = inds_p1.astype(i32)
    r2a = inds_r2[:, 0].astype(i32)
    r2b = inds_r2[:, 1].astype(i32)
    p2 = inds_p2.astype(i32)
    sre = smt_reac.astype(i32)
    spr = smt_prod.astype(i32)
    pk1 = r1 | (p1 << 16)
    pk2 = r2a | (r2b << 10) | (p2 << 20)
    pks = sre | (spr << 16)
    outT = _sc_call(y_in.T, t_in, pk1, jnp.log(alpha1), gamma1,
                    pk2, jnp.log(alpha2), gamma2, pks, k_smt)
    return outT.T
